# baseline (speedup 1.0000x reference)
import jax
import jax.numpy as jnp
from jax import lax
from jax.experimental import pallas as pl
from jax.experimental.pallas import tpu as pltpu
from jax.experimental.pallas import tpu_sc as plsc

N = 10000
E = 320000
D_NODE = 128
D_EDGE = 16
HID = 128
OUT = 128

NC = 2
NS = 16
NW = NC * NS
GL = 128
NG = E // GL
HALF_G = NG // NC
STAGE_G = 125
NUM_STAGES = HALF_G // STAGE_G
CG = NG // NW
CG_EXTRA = NG - NW * CG
NPAD = 10240
CROWS_PAD = NPAD // D_EDGE
IDG = 80


def _sc_body(ef_hbm, ti_hbm, ids_hbm,
             sums_out, counts_out,
             vals_v, idx_v, cidx_v, cx_v, acc_v, cacc_v, id_v, counts_sh,
             sem_v, sem_i):
    c = lax.axis_index("c")
    s = lax.axis_index("s")
    wid = s * NC + c
    rb = s // 8
    r = s % 8

    zvec = jnp.zeros((16,), jnp.float32)

    @plsc.parallel_loop(0, CROWS_PAD, unroll=8)
    def _zero(i):
        acc_v[pl.ds(i * 16, 16)] = zvec
        cacc_v[i, :] = zvec
    pltpu.sync_copy(ids_hbm, id_v)

    @pl.when(s == 0)
    def _():
        pltpu.sync_copy(cacc_v, counts_sh)

    plsc.subcore_barrier()

    def _start(t, b):
        gbase = c * HALF_G + t * STAGE_G
        cp_v = pltpu.async_copy(
            ef_hbm.at[pl.ds(rb, 1), pl.ds(gbase, STAGE_G), pl.ds(r, 1), :],
            vals_v.at[b], sem_v)
        cp_i = pltpu.async_copy(
            ti_hbm.at[pl.ds(gbase, STAGE_G), pl.ds(1, 1), :],
            idx_v.at[b], sem_i)
        return cp_v, cp_i

    pending = _start(0, 0)
    for t in range(NUM_STAGES):
        b = t & 1
        pending[0].wait()
        pending[1].wait()
        if t + 1 < NUM_STAGES:
            pending = _start(t + 1, 1 - b)

        @plsc.parallel_loop(0, STAGE_G, unroll=8)
        def _group(g):
            for k in range(GL // 16):
                tvec = idx_v[b, g, 0, pl.ds(k * 16, 16)]
                vvec = vals_v[b, 0, g, 0, pl.ds(k * 16, 16)]
                plsc.addupdate_scatter(acc_v, [tvec], vvec)

    ones16 = jnp.ones((16,), jnp.float32)
    pltpu.sync_copy(ti_hbm.at[pl.ds(wid * CG, CG), pl.ds(1, 1), :], cidx_v)

    @plsc.parallel_loop(0, CG, unroll=2)
    def _cgroup(g):
        for k in range(GL // 16):
            tvec = cidx_v[g, 0, pl.ds(k * 16, 16)]
            rows = jnp.right_shift(tvec, 4)
            cols = jnp.bitwise_and(tvec, 15)
            plsc.addupdate_scatter(cacc_v, [rows, cols], ones16)

    @pl.when(wid < CG_EXTRA)
    def _():
        pltpu.sync_copy(ti_hbm.at[pl.ds(NW * CG + wid, 1), pl.ds(1, 1), :],
                        cx_v)
        for k in range(GL // 16):
            tvec = cx_v[0, 0, pl.ds(k * 16, 16)]
            rows = jnp.right_shift(tvec, 4)
            cols = jnp.bitwise_and(tvec, 15)
            plsc.addupdate_scatter(cacc_v, [rows, cols], ones16)

    for g in range(CROWS_PAD // IDG):
        pltpu.sync_copy(cacc_v.at[pl.ds(g * IDG, IDG)],
                        counts_sh.at[id_v.at[g]], add=True)

    plsc.subcore_barrier()

    pltpu.sync_copy(acc_v, sums_out.at[c, pl.ds(s * NPAD, NPAD)])

    @pl.when(s == 0)
    def _():
        pltpu.sync_copy(counts_sh, counts_out.at[c])


def _sc_scatter(ef_n, ti_n, ids):
    mesh = plsc.VectorSubcoreMesh(core_axis_name="c", subcore_axis_name="s")
    return pl.kernel(
        _sc_body,
        mesh=mesh,
        out_type=(
            jax.ShapeDtypeStruct((NC, NS * NPAD), jnp.float32),
            jax.ShapeDtypeStruct((NC, CROWS_PAD, D_EDGE), jnp.float32),
        ),
        scratch_types=[
            pltpu.VMEM((2, 1, STAGE_G, 1, GL), jnp.float32),
            pltpu.VMEM((2, STAGE_G, 1, GL), jnp.int32),
            pltpu.VMEM((CG, 1, GL), jnp.int32),
            pltpu.VMEM((1, 1, GL), jnp.int32),
            pltpu.VMEM((NPAD,), jnp.float32),
            pltpu.VMEM((CROWS_PAD, D_EDGE), jnp.float32),
            pltpu.VMEM((CROWS_PAD // IDG, IDG), jnp.int32),
            pltpu.VMEM_SHARED((CROWS_PAD, D_EDGE), jnp.float32),
            pltpu.SemaphoreType.DMA,
            pltpu.SemaphoreType.DMA,
        ],
        compiler_params=pltpu.CompilerParams(use_tc_tiling_on_sc=False,
                                             needs_layout_passes=False),
    )(ef_n, ti_n, ids)


def _mlp_body(sums_ref, counts_ref, node_ref, w1a_ref, w1b_ref, b1_ref,
              w2_ref, b2_ref, out_ref):
    sums = sums_ref[0] + sums_ref[1]
    counts = counts_ref[0] + counts_ref[1]
    agg_t = sums * (1.0 / jnp.maximum(counts, 1.0))[None, :]
    h = lax.dot_general(agg_t, w1a_ref[...], (((0,), (0,)), ((), ())),
                        preferred_element_type=jnp.float32)
    h += jnp.dot(node_ref[...], w1b_ref[...], preferred_element_type=jnp.float32)
    h = jnp.maximum(h + b1_ref[...], 0.0)
    o = jnp.dot(h, w2_ref[...], preferred_element_type=jnp.float32)
    out_ref[...] = o + b2_ref[...]


def _mlp(sums, counts, node_features, w1a, w1b, b1, w2t, b2):
    BN = 1024
    grid = (NPAD // BN,)
    return pl.pallas_call(
        _mlp_body,
        grid=grid,
        in_specs=[
            pl.BlockSpec((NC, D_EDGE, BN), lambda i: (0, 0, i)),
            pl.BlockSpec((NC, BN), lambda i: (0, i)),
            pl.BlockSpec((BN, D_NODE), lambda i: (i, 0)),
            pl.BlockSpec((D_EDGE, HID), lambda i: (0, 0)),
            pl.BlockSpec((D_NODE, HID), lambda i: (0, 0)),
            pl.BlockSpec((1, HID), lambda i: (0, 0)),
            pl.BlockSpec((HID, OUT), lambda i: (0, 0)),
            pl.BlockSpec((1, OUT), lambda i: (0, 0)),
        ],
        out_specs=pl.BlockSpec((BN, OUT), lambda i: (i, 0)),
        out_shape=jax.ShapeDtypeStruct((N, OUT), jnp.float32),
    )(sums, counts, node_features, w1a, w1b, b1, w2t, b2)


def kernel(node_features, edge_index, edge_features, W1, b1, W2, b2):
    ef_n = edge_features.reshape(NG, GL, NC, 8).transpose(2, 0, 3, 1)
    ti_n = edge_index.reshape(2, NG, GL).transpose(1, 0, 2)
    ids = jnp.arange(CROWS_PAD, dtype=jnp.int32).reshape(CROWS_PAD // IDG, IDG)
    sums_flat, counts_grid = _sc_scatter(ef_n, ti_n, ids)
    sums = sums_flat.reshape(NC, NS, NPAD)
    counts2d = counts_grid.reshape(NC, NPAD)
    w1a = W1[:, :D_EDGE].T
    w1b = W1[:, D_EDGE:].T
    w2t = W2.T
    return _mlp(sums, counts2d, node_features, w1a, w1b, b1.reshape(1, HID),
                w2t, b2.reshape(1, OUT))


# speedup vs baseline: 1.0335x; 1.0335x over previous
import jax
import jax.numpy as jnp
from jax import lax
from jax.experimental import pallas as pl
from jax.experimental.pallas import tpu as pltpu
from jax.experimental.pallas import tpu_sc as plsc

N = 10000
E = 320000
D_NODE = 128
D_EDGE = 16
HID = 128
OUT = 128

NC = 2
NS = 16
NW = NC * NS
GL = 128
NG = E // GL
HALF_G = NG // NC
STAGE_G = 125
NUM_STAGES = HALF_G // STAGE_G
CG = NG // NW
CG_EXTRA = NG - NW * CG
NPAD = 10240
CROWS_PAD = NPAD // D_EDGE
IDG = 80


def _sc_body(ef_hbm, ti_hbm, ids_hbm,
             sums_out, counts_out,
             vals_v, idx_v, cidx_v, cx_v, acc_v, cacc_v, id_v, counts_sh,
             sem_v, sem_i):
    c = lax.axis_index("c")
    s = lax.axis_index("s")
    wid = s * NC + c
    rb = s // 8
    r = s % 8

    zvec = jnp.zeros((16,), jnp.float32)

    @plsc.parallel_loop(0, CROWS_PAD, unroll=8)
    def _zero(i):
        acc_v[pl.ds(i * 16, 16)] = zvec
        cacc_v[i, :] = zvec
    pltpu.sync_copy(ids_hbm, id_v)

    @pl.when(s == 0)
    def _():
        pltpu.sync_copy(cacc_v, counts_sh)

    plsc.subcore_barrier()

    def _start(t, b):
        gbase = c * HALF_G + t * STAGE_G
        cp_v = pltpu.async_copy(
            ef_hbm.at[pl.ds(rb, 1), pl.ds(gbase, STAGE_G), pl.ds(r, 1), :],
            vals_v.at[b], sem_v)
        cp_i = pltpu.async_copy(
            ti_hbm.at[pl.ds(gbase, STAGE_G), pl.ds(1, 1), :],
            idx_v.at[b], sem_i)
        return cp_v, cp_i

    pending = _start(0, 0)
    for t in range(NUM_STAGES):
        b = t & 1
        pending[0].wait()
        pending[1].wait()
        if t + 1 < NUM_STAGES:
            pending = _start(t + 1, 1 - b)

        @plsc.parallel_loop(0, STAGE_G, unroll=5)
        def _group(g):
            for k in range(GL // 16):
                tvec = idx_v[b, g, 0, pl.ds(k * 16, 16)]
                vvec = vals_v[b, 0, g, 0, pl.ds(k * 16, 16)]
                plsc.addupdate_scatter(acc_v, [tvec], vvec)

    ones16 = jnp.ones((16,), jnp.float32)
    pltpu.sync_copy(ti_hbm.at[pl.ds(wid * CG, CG), pl.ds(1, 1), :], cidx_v)

    @plsc.parallel_loop(0, CG, unroll=2)
    def _cgroup(g):
        for k in range(GL // 16):
            tvec = cidx_v[g, 0, pl.ds(k * 16, 16)]
            rows = jnp.right_shift(tvec, 4)
            cols = jnp.bitwise_and(tvec, 15)
            plsc.addupdate_scatter(cacc_v, [rows, cols], ones16)

    @pl.when(wid < CG_EXTRA)
    def _():
        pltpu.sync_copy(ti_hbm.at[pl.ds(NW * CG + wid, 1), pl.ds(1, 1), :],
                        cx_v)
        for k in range(GL // 16):
            tvec = cx_v[0, 0, pl.ds(k * 16, 16)]
            rows = jnp.right_shift(tvec, 4)
            cols = jnp.bitwise_and(tvec, 15)
            plsc.addupdate_scatter(cacc_v, [rows, cols], ones16)

    for g in range(CROWS_PAD // IDG):
        pltpu.sync_copy(cacc_v.at[pl.ds(g * IDG, IDG)],
                        counts_sh.at[id_v.at[g]], add=True)

    plsc.subcore_barrier()

    pltpu.sync_copy(acc_v, sums_out.at[c, pl.ds(s * NPAD, NPAD)])

    @pl.when(s == 0)
    def _():
        pltpu.sync_copy(counts_sh, counts_out.at[c])


def _sc_scatter(ef_n, ti_n, ids):
    mesh = plsc.VectorSubcoreMesh(core_axis_name="c", subcore_axis_name="s")
    return pl.kernel(
        _sc_body,
        mesh=mesh,
        out_type=(
            jax.ShapeDtypeStruct((NC, NS * NPAD), jnp.float32),
            jax.ShapeDtypeStruct((NC, CROWS_PAD, D_EDGE), jnp.float32),
        ),
        scratch_types=[
            pltpu.VMEM((2, 1, STAGE_G, 1, GL), jnp.float32),
            pltpu.VMEM((2, STAGE_G, 1, GL), jnp.int32),
            pltpu.VMEM((CG, 1, GL), jnp.int32),
            pltpu.VMEM((1, 1, GL), jnp.int32),
            pltpu.VMEM((NPAD,), jnp.float32),
            pltpu.VMEM((CROWS_PAD, D_EDGE), jnp.float32),
            pltpu.VMEM((CROWS_PAD // IDG, IDG), jnp.int32),
            pltpu.VMEM_SHARED((CROWS_PAD, D_EDGE), jnp.float32),
            pltpu.SemaphoreType.DMA,
            pltpu.SemaphoreType.DMA,
        ],
        compiler_params=pltpu.CompilerParams(use_tc_tiling_on_sc=False,
                                             needs_layout_passes=False),
    )(ef_n, ti_n, ids)


def _mlp_body(sums_ref, counts_ref, node_ref, w1a_ref, w1b_ref, b1_ref,
              w2_ref, b2_ref, out_ref):
    sums = sums_ref[0] + sums_ref[1]
    counts = counts_ref[0] + counts_ref[1]
    agg_t = sums * (1.0 / jnp.maximum(counts, 1.0))[None, :]
    h = lax.dot_general(agg_t, w1a_ref[...], (((0,), (0,)), ((), ())),
                        preferred_element_type=jnp.float32)
    h += jnp.dot(node_ref[...], w1b_ref[...], preferred_element_type=jnp.float32)
    h = jnp.maximum(h + b1_ref[...], 0.0)
    o = jnp.dot(h, w2_ref[...], preferred_element_type=jnp.float32)
    out_ref[...] = o + b2_ref[...]


def _mlp(sums, counts, node_features, w1a, w1b, b1, w2t, b2):
    BN = 1024
    grid = (NPAD // BN,)
    return pl.pallas_call(
        _mlp_body,
        grid=grid,
        in_specs=[
            pl.BlockSpec((NC, D_EDGE, BN), lambda i: (0, 0, i)),
            pl.BlockSpec((NC, BN), lambda i: (0, i)),
            pl.BlockSpec((BN, D_NODE), lambda i: (i, 0)),
            pl.BlockSpec((D_EDGE, HID), lambda i: (0, 0)),
            pl.BlockSpec((D_NODE, HID), lambda i: (0, 0)),
            pl.BlockSpec((1, HID), lambda i: (0, 0)),
            pl.BlockSpec((HID, OUT), lambda i: (0, 0)),
            pl.BlockSpec((1, OUT), lambda i: (0, 0)),
        ],
        out_specs=pl.BlockSpec((BN, OUT), lambda i: (i, 0)),
        out_shape=jax.ShapeDtypeStruct((N, OUT), jnp.float32),
    )(sums, counts, node_features, w1a, w1b, b1, w2t, b2)


def kernel(node_features, edge_index, edge_features, W1, b1, W2, b2):
    ef_n = edge_features.reshape(NG, GL, NC, 8).transpose(2, 0, 3, 1)
    ti_n = edge_index.reshape(2, NG, GL).transpose(1, 0, 2)
    ids = jnp.arange(CROWS_PAD, dtype=jnp.int32).reshape(CROWS_PAD // IDG, IDG)
    sums_flat, counts_grid = _sc_scatter(ef_n, ti_n, ids)
    sums = sums_flat.reshape(NC, NS, NPAD)
    counts2d = counts_grid.reshape(NC, NPAD)
    w1a = W1[:, :D_EDGE].T
    w1b = W1[:, D_EDGE:].T
    w2t = W2.T
    return _mlp(sums, counts2d, node_features, w1a, w1b, b1.reshape(1, HID),
                w2t, b2.reshape(1, OUT))


# speedup vs baseline: 1.0449x; 1.0110x over previous
import jax
import jax.numpy as jnp
from jax import lax
from jax.experimental import pallas as pl
from jax.experimental.pallas import tpu as pltpu
from jax.experimental.pallas import tpu_sc as plsc

N = 10000
E = 320000
D_NODE = 128
D_EDGE = 16
HID = 128
OUT = 128

NC = 2
NS = 16
NW = NC * NS
GL = 128
NG = E // GL
HALF_G = NG // NC
STAGE_G = 125
NUM_STAGES = HALF_G // STAGE_G
CG = NG // NW
CG_EXTRA = NG - NW * CG
NPAD = 10240
CROWS_PAD = NPAD // D_EDGE
IDG = 80


def _sc_body(ef_hbm, ti_hbm, ids_hbm,
             sums_out, counts_out,
             vals_v, idx_v, cidx_v, cx_v, acc_v, cacc_v, id_v, counts_sh,
             sem_v, sem_i):
    c = lax.axis_index("c")
    s = lax.axis_index("s")
    wid = s * NC + c
    rb = s // 8
    r = s % 8

    zvec = jnp.zeros((16,), jnp.float32)

    @plsc.parallel_loop(0, CROWS_PAD, unroll=8)
    def _zero(i):
        acc_v[pl.ds(i * 16, 16)] = zvec
        cacc_v[i, :] = zvec
    pltpu.sync_copy(ids_hbm, id_v)

    @pl.when(s == 0)
    def _():
        pltpu.sync_copy(cacc_v, counts_sh)

    plsc.subcore_barrier()

    def _start(t, b):
        gbase = c * HALF_G + t * STAGE_G
        cp_v = pltpu.async_copy(
            ef_hbm.at[pl.ds(rb, 1), pl.ds(gbase, STAGE_G), pl.ds(r, 1), :],
            vals_v.at[b], sem_v)
        cp_i = pltpu.async_copy(
            ti_hbm.at[pl.ds(gbase, STAGE_G), pl.ds(1, 1), :],
            idx_v.at[b], sem_i)
        return cp_v, cp_i

    pending = _start(0, 0)
    for t in range(NUM_STAGES):
        b = t & 1
        pending[0].wait()
        pending[1].wait()
        if t + 1 < NUM_STAGES:
            pending = _start(t + 1, 1 - b)

        @plsc.parallel_loop(0, STAGE_G, unroll=4)
        def _group(g):
            for k in range(GL // 16):
                tvec = idx_v[b, g, 0, pl.ds(k * 16, 16)]
                vvec = vals_v[b, 0, g, 0, pl.ds(k * 16, 16)]
                plsc.addupdate_scatter(acc_v, [tvec], vvec)

    ones16 = jnp.ones((16,), jnp.float32)
    pltpu.sync_copy(ti_hbm.at[pl.ds(wid * CG, CG), pl.ds(1, 1), :], cidx_v)

    @plsc.parallel_loop(0, CG, unroll=2)
    def _cgroup(g):
        for k in range(GL // 16):
            tvec = cidx_v[g, 0, pl.ds(k * 16, 16)]
            rows = jnp.right_shift(tvec, 4)
            cols = jnp.bitwise_and(tvec, 15)
            plsc.addupdate_scatter(cacc_v, [rows, cols], ones16)

    @pl.when(wid < CG_EXTRA)
    def _():
        pltpu.sync_copy(ti_hbm.at[pl.ds(NW * CG + wid, 1), pl.ds(1, 1), :],
                        cx_v)
        for k in range(GL // 16):
            tvec = cx_v[0, 0, pl.ds(k * 16, 16)]
            rows = jnp.right_shift(tvec, 4)
            cols = jnp.bitwise_and(tvec, 15)
            plsc.addupdate_scatter(cacc_v, [rows, cols], ones16)

    for g in range(CROWS_PAD // IDG):
        pltpu.sync_copy(cacc_v.at[pl.ds(g * IDG, IDG)],
                        counts_sh.at[id_v.at[g]], add=True)

    plsc.subcore_barrier()

    pltpu.sync_copy(acc_v, sums_out.at[c, pl.ds(s * NPAD, NPAD)])

    @pl.when(s == 0)
    def _():
        pltpu.sync_copy(counts_sh, counts_out.at[c])


def _sc_scatter(ef_n, ti_n, ids):
    mesh = plsc.VectorSubcoreMesh(core_axis_name="c", subcore_axis_name="s")
    return pl.kernel(
        _sc_body,
        mesh=mesh,
        out_type=(
            jax.ShapeDtypeStruct((NC, NS * NPAD), jnp.float32),
            jax.ShapeDtypeStruct((NC, CROWS_PAD, D_EDGE), jnp.float32),
        ),
        scratch_types=[
            pltpu.VMEM((2, 1, STAGE_G, 1, GL), jnp.float32),
            pltpu.VMEM((2, STAGE_G, 1, GL), jnp.int32),
            pltpu.VMEM((CG, 1, GL), jnp.int32),
            pltpu.VMEM((1, 1, GL), jnp.int32),
            pltpu.VMEM((NPAD,), jnp.float32),
            pltpu.VMEM((CROWS_PAD, D_EDGE), jnp.float32),
            pltpu.VMEM((CROWS_PAD // IDG, IDG), jnp.int32),
            pltpu.VMEM_SHARED((CROWS_PAD, D_EDGE), jnp.float32),
            pltpu.SemaphoreType.DMA,
            pltpu.SemaphoreType.DMA,
        ],
        compiler_params=pltpu.CompilerParams(use_tc_tiling_on_sc=False,
                                             needs_layout_passes=False),
    )(ef_n, ti_n, ids)


def _hn_body(node_ref, w1b_ref, b1_ref, hn_ref):
    hn_ref[...] = jnp.dot(node_ref[...], w1b_ref[...],
                          preferred_element_type=jnp.float32) + b1_ref[...]


def _hn(node_features, w1b, b1):
    BN = 1024
    return pl.pallas_call(
        _hn_body,
        grid=(NPAD // BN,),
        in_specs=[
            pl.BlockSpec((BN, D_NODE), lambda i: (i, 0)),
            pl.BlockSpec((D_NODE, HID), lambda i: (0, 0)),
            pl.BlockSpec((1, HID), lambda i: (0, 0)),
        ],
        out_specs=pl.BlockSpec((BN, HID), lambda i: (i, 0)),
        out_shape=jax.ShapeDtypeStruct((N, HID), jnp.float32),
    )(node_features, w1b, b1)


def _mlp_body(sums_ref, counts_ref, hn_ref, w1a_ref, w2_ref, b2_ref, out_ref):
    sums = sums_ref[0] + sums_ref[1]
    counts = counts_ref[0] + counts_ref[1]
    agg_t = sums * (1.0 / jnp.maximum(counts, 1.0))[None, :]
    h = lax.dot_general(agg_t, w1a_ref[...], (((0,), (0,)), ((), ())),
                        preferred_element_type=jnp.float32)
    h = jnp.maximum(h + hn_ref[...], 0.0)
    o = jnp.dot(h, w2_ref[...], preferred_element_type=jnp.float32)
    out_ref[...] = o + b2_ref[...]


def _mlp(sums, counts, hn, w1a, w2t, b2):
    BN = 1024
    grid = (NPAD // BN,)
    return pl.pallas_call(
        _mlp_body,
        grid=grid,
        in_specs=[
            pl.BlockSpec((NC, D_EDGE, BN), lambda i: (0, 0, i)),
            pl.BlockSpec((NC, BN), lambda i: (0, i)),
            pl.BlockSpec((BN, HID), lambda i: (i, 0)),
            pl.BlockSpec((D_EDGE, HID), lambda i: (0, 0)),
            pl.BlockSpec((HID, OUT), lambda i: (0, 0)),
            pl.BlockSpec((1, OUT), lambda i: (0, 0)),
        ],
        out_specs=pl.BlockSpec((BN, OUT), lambda i: (i, 0)),
        out_shape=jax.ShapeDtypeStruct((N, OUT), jnp.float32),
    )(sums, counts, hn, w1a, w2t, b2)


def kernel(node_features, edge_index, edge_features, W1, b1, W2, b2):
    ef_n = edge_features.reshape(NG, GL, NC, 8).transpose(2, 0, 3, 1)
    ti_n = edge_index.reshape(2, NG, GL).transpose(1, 0, 2)
    ids = jnp.arange(CROWS_PAD, dtype=jnp.int32).reshape(CROWS_PAD // IDG, IDG)
    sums_flat, counts_grid = _sc_scatter(ef_n, ti_n, ids)
    sums = sums_flat.reshape(NC, NS, NPAD)
    counts2d = counts_grid.reshape(NC, NPAD)
    w1a = W1[:, :D_EDGE].T
    w1b = W1[:, D_EDGE:].T
    w2t = W2.T
    hn = _hn(node_features, w1b, b1.reshape(1, HID))
    return _mlp(sums, counts2d, hn, w1a, w2t, b2.reshape(1, OUT))


# speedup vs baseline: 1.1116x; 1.0638x over previous
import jax
import jax.numpy as jnp
import numpy as np
from jax import lax
from jax.experimental import pallas as pl
from jax.experimental.pallas import tpu as pltpu
from jax.experimental.pallas import tpu_sc as plsc

N = 10000
E = 320000
D_NODE = 128
D_EDGE = 16
HID = 128
OUT = 128

NC = 2
NS = 16
NW = NC * NS
GL = 128
NG = E // GL
HALF_G = NG // NC
STAGE_G = 125
NUM_STAGES = HALF_G // STAGE_G
CG = NG // NW
CG_EXTRA = NG - NW * CG
NPAD = 10240
CROWS_PAD = NPAD // D_EDGE
IDG = 80


def _sc_body(ef_hbm, ti_hbm, ids_hbm,
             sums_out, counts_out,
             vals_v, idx_v, cidx_v, cx_v, acc_v, cacc_v, id_v, counts_sh,
             sem_v, sem_i, sem_c):
    c = lax.axis_index("c")
    s = lax.axis_index("s")
    wid = s * NC + c
    rb = s // 8
    r = s % 8

    def _start(t, b):
        gbase = c * HALF_G + t * STAGE_G
        cp_v = pltpu.async_copy(
            ef_hbm.at[pl.ds(rb, 1), pl.ds(gbase, STAGE_G), pl.ds(r, 1), :],
            vals_v.at[b], sem_v)
        cp_i = pltpu.async_copy(
            ti_hbm.at[pl.ds(gbase, STAGE_G), pl.ds(1, 1), :],
            idx_v.at[b], sem_i)
        return cp_v, cp_i

    pending = _start(0, 0)
    cp_c = pltpu.async_copy(ti_hbm.at[pl.ds(wid * CG, CG), pl.ds(1, 1), :],
                            cidx_v, sem_c)

    zvec = jnp.zeros((16,), jnp.float32)

    @plsc.parallel_loop(0, CROWS_PAD, unroll=8)
    def _zero(i):
        acc_v[pl.ds(i * 16, 16)] = zvec
        cacc_v[i, :] = zvec
    pltpu.sync_copy(ids_hbm, id_v)

    @pl.when(s == 0)
    def _():
        pltpu.sync_copy(cacc_v, counts_sh)

    plsc.subcore_barrier()

    for t in range(NUM_STAGES):
        b = t & 1
        pending[0].wait()
        pending[1].wait()
        if t + 1 < NUM_STAGES:
            pending = _start(t + 1, 1 - b)

        @plsc.parallel_loop(0, STAGE_G, unroll=4)
        def _group(g):
            for k in range(GL // 16):
                tvec = idx_v[b, g, 0, pl.ds(k * 16, 16)]
                vvec = vals_v[b, 0, g, 0, pl.ds(k * 16, 16)]
                plsc.addupdate_scatter(acc_v, [tvec], vvec)

    ones16 = jnp.ones((16,), jnp.float32)
    cp_c.wait()

    @plsc.parallel_loop(0, CG, unroll=2)
    def _cgroup(g):
        for k in range(GL // 16):
            tvec = cidx_v[g, 0, pl.ds(k * 16, 16)]
            rows = jnp.right_shift(tvec, 4)
            cols = jnp.bitwise_and(tvec, 15)
            plsc.addupdate_scatter(cacc_v, [rows, cols], ones16)

    @pl.when(wid < CG_EXTRA)
    def _():
        pltpu.sync_copy(ti_hbm.at[pl.ds(NW * CG + wid, 1), pl.ds(1, 1), :],
                        cx_v)
        for k in range(GL // 16):
            tvec = cx_v[0, 0, pl.ds(k * 16, 16)]
            rows = jnp.right_shift(tvec, 4)
            cols = jnp.bitwise_and(tvec, 15)
            plsc.addupdate_scatter(cacc_v, [rows, cols], ones16)

    for g in range(CROWS_PAD // IDG):
        pltpu.sync_copy(cacc_v.at[pl.ds(g * IDG, IDG)],
                        counts_sh.at[id_v.at[g]], add=True)

    plsc.subcore_barrier()

    pltpu.sync_copy(acc_v, sums_out.at[c, pl.ds(s * NPAD, NPAD)])

    @pl.when(s == 0)
    def _():
        pltpu.sync_copy(counts_sh, counts_out.at[c])


def _sc_scatter(ef_n, ti_n, ids):
    mesh = plsc.VectorSubcoreMesh(core_axis_name="c", subcore_axis_name="s")
    return pl.kernel(
        _sc_body,
        mesh=mesh,
        out_type=(
            jax.ShapeDtypeStruct((NC, NS * NPAD), jnp.float32),
            jax.ShapeDtypeStruct((NC, CROWS_PAD, D_EDGE), jnp.float32),
        ),
        scratch_types=[
            pltpu.VMEM((2, 1, STAGE_G, 1, GL), jnp.float32),
            pltpu.VMEM((2, STAGE_G, 1, GL), jnp.int32),
            pltpu.VMEM((CG, 1, GL), jnp.int32),
            pltpu.VMEM((1, 1, GL), jnp.int32),
            pltpu.VMEM((NPAD,), jnp.float32),
            pltpu.VMEM((CROWS_PAD, D_EDGE), jnp.float32),
            pltpu.VMEM((CROWS_PAD // IDG, IDG), jnp.int32),
            pltpu.VMEM_SHARED((CROWS_PAD, D_EDGE), jnp.float32),
            pltpu.SemaphoreType.DMA,
            pltpu.SemaphoreType.DMA,
            pltpu.SemaphoreType.DMA,
        ],
        compiler_params=pltpu.CompilerParams(use_tc_tiling_on_sc=False,
                                             needs_layout_passes=False),
    )(ef_n, ti_n, ids)


def _hn_body(node_ref, w1b_ref, b1_ref, hn_ref):
    hn_ref[...] = jnp.dot(node_ref[...], w1b_ref[...],
                          preferred_element_type=jnp.float32) + b1_ref[...]


def _hn(node_features, w1b, b1):
    BN = 2048
    return pl.pallas_call(
        _hn_body,
        grid=(NPAD // BN,),
        in_specs=[
            pl.BlockSpec((BN, D_NODE), lambda i: (i, 0)),
            pl.BlockSpec((D_NODE, HID), lambda i: (0, 0)),
            pl.BlockSpec((1, HID), lambda i: (0, 0)),
        ],
        out_specs=pl.BlockSpec((BN, HID), lambda i: (i, 0)),
        out_shape=jax.ShapeDtypeStruct((N, HID), jnp.float32),
    )(node_features, w1b, b1)


def _mlp_body(sums_ref, counts_ref, hn_ref, w1a_ref, w2_ref, b2_ref, out_ref):
    sums = sums_ref[0] + sums_ref[1]
    counts = counts_ref[0] + counts_ref[1]
    agg_t = sums * (1.0 / jnp.maximum(counts, 1.0))[None, :]
    h = lax.dot_general(agg_t, w1a_ref[...], (((0,), (0,)), ((), ())),
                        preferred_element_type=jnp.float32)
    h = jnp.maximum(h + hn_ref[...], 0.0)
    o = jnp.dot(h, w2_ref[...], preferred_element_type=jnp.float32)
    out_ref[...] = o + b2_ref[...]


def _mlp(sums, counts, hn, w1a, w2t, b2):
    BN = 2048
    grid = (NPAD // BN,)
    return pl.pallas_call(
        _mlp_body,
        grid=grid,
        in_specs=[
            pl.BlockSpec((NC, D_EDGE, BN), lambda i: (0, 0, i)),
            pl.BlockSpec((NC, BN), lambda i: (0, i)),
            pl.BlockSpec((BN, HID), lambda i: (i, 0)),
            pl.BlockSpec((D_EDGE, HID), lambda i: (0, 0)),
            pl.BlockSpec((HID, OUT), lambda i: (0, 0)),
            pl.BlockSpec((1, OUT), lambda i: (0, 0)),
        ],
        out_specs=pl.BlockSpec((BN, OUT), lambda i: (i, 0)),
        out_shape=jax.ShapeDtypeStruct((N, OUT), jnp.float32),
    )(sums, counts, hn, w1a, w2t, b2)


def kernel(node_features, edge_index, edge_features, W1, b1, W2, b2):
    ef_n = edge_features.reshape(NG, GL, NC, 8).transpose(2, 0, 3, 1)
    ti_n = edge_index.reshape(2, NG, GL).transpose(1, 0, 2)
    ids = np.arange(CROWS_PAD, dtype=np.int32).reshape(CROWS_PAD // IDG, IDG)
    sums_flat, counts_grid = _sc_scatter(ef_n, ti_n, ids)
    sums = sums_flat.reshape(NC, NS, NPAD)
    counts2d = counts_grid.reshape(NC, NPAD)
    w1a = W1[:, :D_EDGE].T
    w1b = W1[:, D_EDGE:].T
    w2t = W2.T
    hn = _hn(node_features, w1b, b1.reshape(1, HID))
    return _mlp(sums, counts2d, hn, w1a, w2t, b2.reshape(1, OUT))


# speedup vs baseline: 1.1430x; 1.0283x over previous
import jax
import jax.numpy as jnp
import numpy as np
from jax import lax
from jax.experimental import pallas as pl
from jax.experimental.pallas import tpu as pltpu
from jax.experimental.pallas import tpu_sc as plsc

N = 10000
E = 320000
D_NODE = 128
D_EDGE = 16
HID = 128
OUT = 128

NC = 2
NS = 16
NW = NC * NS
GL = 128
NG = E // GL
HALF_G = NG // NC
STAGE_G = 125
NUM_STAGES = HALF_G // STAGE_G
CG = NG // NW
CG_EXTRA = NG - NW * CG
NPAD = 10240


def _sc_body(ef_hbm, ti_hbm,
             sums_out, counts_out,
             vals_v, idx_v, cidx_v, cx_v, acc_v, cacc_v,
             sem_v, sem_i, sem_c):
    c = lax.axis_index("c")
    s = lax.axis_index("s")
    wid = s * NC + c
    rb = s // 8
    r = s % 8

    def _start(t, b):
        gbase = c * HALF_G + t * STAGE_G
        cp_v = pltpu.async_copy(
            ef_hbm.at[pl.ds(rb, 1), pl.ds(gbase, STAGE_G), pl.ds(r, 1), :],
            vals_v.at[b], sem_v)
        cp_i = pltpu.async_copy(
            ti_hbm.at[pl.ds(gbase, STAGE_G), pl.ds(1, 1), :],
            idx_v.at[b], sem_i)
        return cp_v, cp_i

    pending = _start(0, 0)
    cp_c = pltpu.async_copy(ti_hbm.at[pl.ds(wid * CG, CG), pl.ds(1, 1), :],
                            cidx_v, sem_c)

    zvec = jnp.zeros((16,), jnp.float32)

    @plsc.parallel_loop(0, NPAD // 16, unroll=8)
    def _zero(i):
        acc_v[pl.ds(i * 16, 16)] = zvec
        cacc_v[pl.ds(i * 16, 16)] = zvec

    for t in range(NUM_STAGES):
        b = t & 1
        pending[0].wait()
        pending[1].wait()
        if t + 1 < NUM_STAGES:
            pending = _start(t + 1, 1 - b)

        @plsc.parallel_loop(0, STAGE_G, unroll=4)
        def _group(g):
            for k in range(GL // 16):
                tvec = idx_v[b, g, 0, pl.ds(k * 16, 16)]
                vvec = vals_v[b, 0, g, 0, pl.ds(k * 16, 16)]
                plsc.addupdate_scatter(acc_v, [tvec], vvec)

    ones16 = jnp.ones((16,), jnp.float32)
    cp_c.wait()

    @plsc.parallel_loop(0, CG, unroll=2)
    def _cgroup(g):
        for k in range(GL // 16):
            tvec = cidx_v[g, 0, pl.ds(k * 16, 16)]
            plsc.addupdate_scatter(cacc_v, [tvec], ones16)

    @pl.when(wid < CG_EXTRA)
    def _():
        pltpu.sync_copy(ti_hbm.at[pl.ds(NW * CG + wid, 1), pl.ds(1, 1), :],
                        cx_v)
        for k in range(GL // 16):
            tvec = cx_v[0, 0, pl.ds(k * 16, 16)]
            plsc.addupdate_scatter(cacc_v, [tvec], ones16)

    pltpu.sync_copy(acc_v, sums_out.at[c, pl.ds(s * NPAD, NPAD)])
    pltpu.sync_copy(cacc_v, counts_out.at[c, pl.ds(s * NPAD, NPAD)])


def _sc_scatter(ef_n, ti_n):
    mesh = plsc.VectorSubcoreMesh(core_axis_name="c", subcore_axis_name="s")
    return pl.kernel(
        _sc_body,
        mesh=mesh,
        out_type=(
            jax.ShapeDtypeStruct((NC, NS * NPAD), jnp.float32),
            jax.ShapeDtypeStruct((NC, NS * NPAD), jnp.float32),
        ),
        scratch_types=[
            pltpu.VMEM((2, 1, STAGE_G, 1, GL), jnp.float32),
            pltpu.VMEM((2, STAGE_G, 1, GL), jnp.int32),
            pltpu.VMEM((CG, 1, GL), jnp.int32),
            pltpu.VMEM((1, 1, GL), jnp.int32),
            pltpu.VMEM((NPAD,), jnp.float32),
            pltpu.VMEM((NPAD,), jnp.float32),
            pltpu.SemaphoreType.DMA,
            pltpu.SemaphoreType.DMA,
            pltpu.SemaphoreType.DMA,
        ],
        compiler_params=pltpu.CompilerParams(use_tc_tiling_on_sc=False,
                                             needs_layout_passes=False),
    )(ef_n, ti_n)


def _hn_body(node_ref, w1b_ref, b1_ref, hn_ref):
    hn_ref[...] = jnp.dot(node_ref[...], w1b_ref[...],
                          preferred_element_type=jnp.float32) + b1_ref[...]


def _hn(node_features, w1b, b1):
    BN = 2048
    return pl.pallas_call(
        _hn_body,
        grid=(NPAD // BN,),
        in_specs=[
            pl.BlockSpec((BN, D_NODE), lambda i: (i, 0)),
            pl.BlockSpec((D_NODE, HID), lambda i: (0, 0)),
            pl.BlockSpec((1, HID), lambda i: (0, 0)),
        ],
        out_specs=pl.BlockSpec((BN, HID), lambda i: (i, 0)),
        out_shape=jax.ShapeDtypeStruct((N, HID), jnp.float32),
    )(node_features, w1b, b1)


def _mlp_body(sums_ref, counts_ref, hn_ref, w1a_ref, w2_ref, b2_ref, out_ref):
    sums = sums_ref[0] + sums_ref[1]
    counts = jnp.sum(counts_ref[...], axis=0)
    agg_t = sums * (1.0 / jnp.maximum(counts, 1.0))[None, :]
    h = lax.dot_general(agg_t, w1a_ref[...], (((0,), (0,)), ((), ())),
                        preferred_element_type=jnp.float32)
    h = jnp.maximum(h + hn_ref[...], 0.0)
    o = jnp.dot(h, w2_ref[...], preferred_element_type=jnp.float32)
    out_ref[...] = o + b2_ref[...]


def _mlp(sums, counts, hn, w1a, w2t, b2):
    BN = 2048
    grid = (NPAD // BN,)
    return pl.pallas_call(
        _mlp_body,
        grid=grid,
        in_specs=[
            pl.BlockSpec((NC, D_EDGE, BN), lambda i: (0, 0, i)),
            pl.BlockSpec((NW, BN), lambda i: (0, i)),
            pl.BlockSpec((BN, HID), lambda i: (i, 0)),
            pl.BlockSpec((D_EDGE, HID), lambda i: (0, 0)),
            pl.BlockSpec((HID, OUT), lambda i: (0, 0)),
            pl.BlockSpec((1, OUT), lambda i: (0, 0)),
        ],
        out_specs=pl.BlockSpec((BN, OUT), lambda i: (i, 0)),
        out_shape=jax.ShapeDtypeStruct((N, OUT), jnp.float32),
    )(sums, counts, hn, w1a, w2t, b2)


def kernel(node_features, edge_index, edge_features, W1, b1, W2, b2):
    ef_n = edge_features.reshape(NG, GL, NC, 8).transpose(2, 0, 3, 1)
    ti_n = edge_index.reshape(2, NG, GL).transpose(1, 0, 2)
    sums_flat, counts_flat = _sc_scatter(ef_n, ti_n)
    sums = sums_flat.reshape(NC, NS, NPAD)
    counts32 = counts_flat.reshape(NW, NPAD)
    w1a = W1[:, :D_EDGE].T
    w1b = W1[:, D_EDGE:].T
    w2t = W2.T
    hn = _hn(node_features, w1b, b1.reshape(1, HID))
    return _mlp(sums, counts32, hn, w1a, w2t, b2.reshape(1, OUT))
